# baseline (speedup 1.0000x reference)
import jax
import jax.numpy as jnp
from jax.experimental import pallas as pl
from jax.experimental.pallas import tpu as pltpu

_L = 50
_B = 4096
_D = 128


def _body(idx_ref, cc_ref, x_ref, buf_ref, out_ref, nbuf_ref):
    i = pl.program_id(0)
    is_idx = i == idx_ref[0]

    @pl.when(is_idx)
    def _write_slot():
        nbuf_ref[0] = x_ref[...]
        out_ref[...] = jnp.where(cc_ref[0] >= _L, buf_ref[0],
                                 jnp.zeros_like(buf_ref[0]))

    @pl.when(jnp.logical_not(is_idx))
    def _copy_row():
        nbuf_ref[0] = buf_ref[0]


def kernel(x, buffer, index, call_count):
    idx = jnp.asarray(index, jnp.int32).reshape(1)
    cc = jnp.asarray(call_count, jnp.int32).reshape(1)
    grid_spec = pltpu.PrefetchScalarGridSpec(
        num_scalar_prefetch=2,
        grid=(_L,),
        in_specs=[
            pl.BlockSpec((_B, _D), lambda i, *_: (0, 0)),
            pl.BlockSpec((1, _B, _D), lambda i, *_: (i, 0, 0)),
        ],
        out_specs=[
            pl.BlockSpec((_B, _D), lambda i, *_: (0, 0)),
            pl.BlockSpec((1, _B, _D), lambda i, *_: (i, 0, 0)),
        ],
    )
    output, new_buffer = pl.pallas_call(
        _body,
        grid_spec=grid_spec,
        out_shape=(
            jax.ShapeDtypeStruct((_B, _D), x.dtype),
            jax.ShapeDtypeStruct((_L, _B, _D), buffer.dtype),
        ),
    )(idx, cc, x, buffer)
    return output, new_buffer
